# baseline (speedup 1.0000x reference)
import math

import jax
import jax.numpy as jnp
from jax.experimental import pallas as pl
from jax.experimental.pallas import tpu as pltpu

_B, _Z, _D, _OUT = 64, 1024, 1024, 256
_NQ = 4
_ZQ = _Z // _NQ


def _zoner_body(txt_ref, zq0, zq1, zq2, zq3, wt_ref, bt_ref, wz_ref, bz_ref,
                out_ref, t_ref):
    b = pl.program_id(0)

    @pl.when(b == 0)
    def _():
        t_ref[...] = jnp.tanh(
            jnp.dot(txt_ref[...], wt_ref[...],
                    preferred_element_type=jnp.float32) + bt_ref[...])

    t_row = t_ref[pl.ds(b, 1), :]
    tmat = jnp.broadcast_to(jnp.transpose(t_row), (_OUT, 128))
    parts = []
    for zq in (zq0, zq1, zq2, zq3):
        z = jnp.tanh(
            jnp.dot(zq[0], wz_ref[...],
                    preferred_element_type=jnp.float32) + bz_ref[...])
        r = jax.lax.dot_general(z, tmat, (((1,), (0,)), ((), ())),
                                preferred_element_type=jnp.float32)
        parts.append(jnp.transpose(jax.lax.slice(r, (0, 0), (_ZQ, 1))))
    s = jnp.concatenate(parts, axis=1) * (1.0 / math.sqrt(_D))
    m = jnp.max(s, axis=1, keepdims=True)
    e = jnp.exp(s - m)
    out_ref[0] = e / jnp.sum(e, axis=1, keepdims=True)


def kernel(txt_embeds, zone_embeds, W_txt, b_txt, W_zone, b_zone):
    bt = b_txt.reshape(1, _OUT)
    bz = b_zone.reshape(1, _OUT)
    zone_specs = [
        pl.BlockSpec((1, _ZQ, _D), lambda b, q=q: (b, q, 0))
        for q in range(_NQ)
    ]
    return pl.pallas_call(
        _zoner_body,
        grid=(_B,),
        in_specs=[
            pl.BlockSpec((_B, _D), lambda b: (0, 0)),
            *zone_specs,
            pl.BlockSpec((_D, _OUT), lambda b: (0, 0)),
            pl.BlockSpec((1, _OUT), lambda b: (0, 0)),
            pl.BlockSpec((_D, _OUT), lambda b: (0, 0)),
            pl.BlockSpec((1, _OUT), lambda b: (0, 0)),
        ],
        out_specs=pl.BlockSpec((1, 1, _Z), lambda b: (b, 0, 0)),
        out_shape=jax.ShapeDtypeStruct((_B, 1, _Z), jnp.float32),
        scratch_shapes=[pltpu.VMEM((_B, _OUT), jnp.float32)],
    )(txt_embeds, zone_embeds, zone_embeds, zone_embeds, zone_embeds,
      W_txt, bt, W_zone, bz).reshape(_B, _Z)


# speedup vs baseline: 1.1475x; 1.1475x over previous
import math

import jax
import jax.numpy as jnp
from jax.experimental import pallas as pl
from jax.experimental.pallas import tpu as pltpu

_B, _Z, _D, _OUT = 64, 1024, 1024, 256


def _zoner_body(txt_ref, zone_ref, wt_ref, bt_ref, wz_ref, bz_ref,
                out_ref, t_ref):
    b = pl.program_id(0)

    @pl.when(b == 0)
    def _():
        t_ref[...] = jnp.tanh(
            jnp.dot(txt_ref[...].astype(jnp.bfloat16), wt_ref[...],
                    preferred_element_type=jnp.float32) + bt_ref[...])

    z = jnp.tanh(
        jnp.dot(zone_ref[0].astype(jnp.bfloat16), wz_ref[...],
                preferred_element_type=jnp.float32) + bz_ref[...])
    t_row = t_ref[pl.ds(b, 1), :]
    tmat = jnp.broadcast_to(jnp.transpose(t_row), (_OUT, 128)).astype(jnp.bfloat16)
    r = jax.lax.dot_general(z.astype(jnp.bfloat16), tmat,
                            (((1,), (0,)), ((), ())),
                            preferred_element_type=jnp.float32)
    logits = jax.lax.slice(r, (0, 0), (_Z, 1))
    s = jnp.transpose(logits) * (1.0 / math.sqrt(_D))
    m = jnp.max(s, axis=1, keepdims=True)
    e = jnp.exp(s - m)
    out_ref[0] = e / jnp.sum(e, axis=1, keepdims=True)


def kernel(txt_embeds, zone_embeds, W_txt, b_txt, W_zone, b_zone):
    bt = b_txt.reshape(1, _OUT)
    bz = b_zone.reshape(1, _OUT)
    return pl.pallas_call(
        _zoner_body,
        grid=(_B,),
        in_specs=[
            pl.BlockSpec((_B, _D), lambda b: (0, 0)),
            pl.BlockSpec((1, _Z, _D), lambda b: (b, 0, 0)),
            pl.BlockSpec((_D, _OUT), lambda b: (0, 0)),
            pl.BlockSpec((1, _OUT), lambda b: (0, 0)),
            pl.BlockSpec((_D, _OUT), lambda b: (0, 0)),
            pl.BlockSpec((1, _OUT), lambda b: (0, 0)),
        ],
        out_specs=pl.BlockSpec((1, 1, _Z), lambda b: (b, 0, 0)),
        out_shape=jax.ShapeDtypeStruct((_B, 1, _Z), jnp.float32),
        scratch_shapes=[pltpu.VMEM((_B, _OUT), jnp.float32)],
    )(txt_embeds, zone_embeds,
      W_txt.astype(jnp.bfloat16), bt,
      W_zone.astype(jnp.bfloat16), bz).reshape(_B, _Z)


# speedup vs baseline: 1.1954x; 1.0418x over previous
import math

import jax
import jax.numpy as jnp
from jax.experimental import pallas as pl
from jax.experimental.pallas import tpu as pltpu

_B, _Z, _D, _OUT = 64, 1024, 1024, 256


def _zoner_body(txt_ref, zone_ref, wt_ref, bt_ref, wz_ref, bz_ref,
                out_ref, t_ref):
    b = pl.program_id(0)

    @pl.when(b == 0)
    def _():
        t_ref[...] = jnp.tanh(
            jnp.dot(txt_ref[...].astype(jnp.bfloat16), wt_ref[...],
                    preferred_element_type=jnp.float32)
            + bt_ref[...]) * (1.0 / math.sqrt(_D))

    tb = t_ref[pl.ds(b, 1), :].astype(jnp.bfloat16)
    z = jnp.tanh(
        jnp.dot(zone_ref[0].astype(jnp.bfloat16), wz_ref[...],
                preferred_element_type=jnp.float32) + bz_ref[...])
    s = jax.lax.dot_general(tb, z.astype(jnp.bfloat16),
                            (((1,), (1,)), ((), ())),
                            preferred_element_type=jnp.float32)
    e = jnp.exp(s)
    out_ref[0] = e / jnp.sum(e, axis=1, keepdims=True)


def kernel(txt_embeds, zone_embeds, W_txt, b_txt, W_zone, b_zone):
    bt = b_txt.reshape(1, _OUT)
    bz = b_zone.reshape(1, _OUT)
    return pl.pallas_call(
        _zoner_body,
        grid=(_B,),
        in_specs=[
            pl.BlockSpec((_B, _D), lambda b: (0, 0)),
            pl.BlockSpec((1, _Z, _D), lambda b: (b, 0, 0)),
            pl.BlockSpec((_D, _OUT), lambda b: (0, 0)),
            pl.BlockSpec((1, _OUT), lambda b: (0, 0)),
            pl.BlockSpec((_D, _OUT), lambda b: (0, 0)),
            pl.BlockSpec((1, _OUT), lambda b: (0, 0)),
        ],
        out_specs=pl.BlockSpec((1, 1, _Z), lambda b: (b, 0, 0)),
        out_shape=jax.ShapeDtypeStruct((_B, 1, _Z), jnp.float32),
        scratch_shapes=[pltpu.VMEM((_B, _OUT), jnp.float32)],
    )(txt_embeds, zone_embeds,
      W_txt.astype(jnp.bfloat16), bt,
      W_zone.astype(jnp.bfloat16), bz).reshape(_B, _Z)


# speedup vs baseline: 1.2900x; 1.0791x over previous
import math

import jax
import jax.numpy as jnp
from jax.experimental import pallas as pl
from jax.experimental.pallas import tpu as pltpu

_B, _Z, _D, _OUT = 64, 1024, 1024, 256
_RB = 2


def _zoner_body(txt_ref, zone_ref, wt_ref, bt_ref, wz_ref, bz_ref,
                out_ref, t_ref):
    b = pl.program_id(0)

    @pl.when(b == 0)
    def _():
        t_ref[...] = jnp.tanh(
            jnp.dot(txt_ref[...].astype(jnp.bfloat16), wt_ref[...],
                    preferred_element_type=jnp.float32)
            + bt_ref[...]) * (1.0 / math.sqrt(_D))

    for r in range(_RB):
        tb = t_ref[pl.ds(b * _RB + r, 1), :].astype(jnp.bfloat16)
        z = jnp.tanh(
            jnp.dot(zone_ref[r].astype(jnp.bfloat16), wz_ref[...],
                    preferred_element_type=jnp.float32) + bz_ref[...])
        s = jax.lax.dot_general(tb, z.astype(jnp.bfloat16),
                                (((1,), (1,)), ((), ())),
                                preferred_element_type=jnp.float32)
        e = jnp.exp(s)
        out_ref[r] = e / jnp.sum(e, axis=1, keepdims=True)


def kernel(txt_embeds, zone_embeds, W_txt, b_txt, W_zone, b_zone):
    bt = b_txt.reshape(1, _OUT)
    bz = b_zone.reshape(1, _OUT)
    return pl.pallas_call(
        _zoner_body,
        grid=(_B // _RB,),
        in_specs=[
            pl.BlockSpec((_B, _D), lambda b: (0, 0)),
            pl.BlockSpec((_RB, _Z, _D), lambda b: (b, 0, 0)),
            pl.BlockSpec((_D, _OUT), lambda b: (0, 0)),
            pl.BlockSpec((1, _OUT), lambda b: (0, 0)),
            pl.BlockSpec((_D, _OUT), lambda b: (0, 0)),
            pl.BlockSpec((1, _OUT), lambda b: (0, 0)),
        ],
        out_specs=pl.BlockSpec((_RB, 1, _Z), lambda b: (b, 0, 0)),
        out_shape=jax.ShapeDtypeStruct((_B, 1, _Z), jnp.float32),
        scratch_shapes=[pltpu.VMEM((_B, _OUT), jnp.float32)],
    )(txt_embeds, zone_embeds,
      W_txt.astype(jnp.bfloat16), bt,
      W_zone.astype(jnp.bfloat16), bz).reshape(_B, _Z)


# speedup vs baseline: 1.3953x; 1.0817x over previous
import math

import jax
import jax.numpy as jnp
from jax.experimental import pallas as pl
from jax.experimental.pallas import tpu as pltpu

_B, _Z, _D, _OUT = 64, 1024, 1024, 256
_RB = 4


def _zoner_body(txt_ref, zone_ref, wt_ref, bt_ref, wz_ref, bz_ref,
                out_ref, t_ref):
    b = pl.program_id(0)

    @pl.when(b == 0)
    def _():
        t_ref[...] = jnp.tanh(
            jnp.dot(txt_ref[...].astype(jnp.bfloat16), wt_ref[...],
                    preferred_element_type=jnp.float32)
            + bt_ref[...]) * (1.0 / math.sqrt(_D))

    for r in range(_RB):
        tb = t_ref[pl.ds(b * _RB + r, 1), :].astype(jnp.bfloat16)
        z = jnp.tanh(
            jnp.dot(zone_ref[r].astype(jnp.bfloat16), wz_ref[...],
                    preferred_element_type=jnp.float32) + bz_ref[...])
        s = jax.lax.dot_general(tb, z.astype(jnp.bfloat16),
                                (((1,), (1,)), ((), ())),
                                preferred_element_type=jnp.float32)
        e = jnp.exp(s)
        out_ref[r] = e / jnp.sum(e, axis=1, keepdims=True)


def kernel(txt_embeds, zone_embeds, W_txt, b_txt, W_zone, b_zone):
    bt = b_txt.reshape(1, _OUT)
    bz = b_zone.reshape(1, _OUT)
    return pl.pallas_call(
        _zoner_body,
        grid=(_B // _RB,),
        in_specs=[
            pl.BlockSpec((_B, _D), lambda b: (0, 0)),
            pl.BlockSpec((_RB, _Z, _D), lambda b: (b, 0, 0)),
            pl.BlockSpec((_D, _OUT), lambda b: (0, 0)),
            pl.BlockSpec((1, _OUT), lambda b: (0, 0)),
            pl.BlockSpec((_D, _OUT), lambda b: (0, 0)),
            pl.BlockSpec((1, _OUT), lambda b: (0, 0)),
        ],
        out_specs=pl.BlockSpec((_RB, 1, _Z), lambda b: (b, 0, 0)),
        out_shape=jax.ShapeDtypeStruct((_B, 1, _Z), jnp.float32),
        scratch_shapes=[pltpu.VMEM((_B, _OUT), jnp.float32)],
    )(txt_embeds, zone_embeds,
      W_txt.astype(jnp.bfloat16), bt,
      W_zone.astype(jnp.bfloat16), bz).reshape(_B, _Z)


# speedup vs baseline: 1.5953x; 1.1433x over previous
import math

import jax
import jax.numpy as jnp
from jax.experimental import pallas as pl
from jax.experimental.pallas import tpu as pltpu

_B, _Z, _D, _OUT = 64, 1024, 1024, 256
_NBUF = 3


def _zoner_body(txt_ref, zone_hbm, wt_ref, bt_ref, wz_ref, bz_ref,
                out_ref, t_ref, zbuf, sems):
    b = pl.program_id(0)

    def _copy(row):
        slot = jax.lax.rem(row, _NBUF)
        return pltpu.make_async_copy(
            zone_hbm.at[pl.ds(row, 1)],
            zbuf.at[pl.ds(slot, 1)],
            sems.at[slot])

    @pl.when(b == 0)
    def _():
        for k in range(_NBUF - 1):
            _copy(k).start()
        t_ref[...] = jnp.tanh(
            jnp.dot(txt_ref[...].astype(jnp.bfloat16), wt_ref[...],
                    preferred_element_type=jnp.float32)
            + bt_ref[...]) * (1.0 / math.sqrt(_D))

    @pl.when(b + _NBUF - 1 < _B)
    def _():
        _copy(b + _NBUF - 1).start()

    _copy(b).wait()
    zrow = zbuf[pl.ds(jax.lax.rem(b, _NBUF), 1)][0]
    tb = t_ref[pl.ds(b, 1), :].astype(jnp.bfloat16)
    z = jnp.tanh(
        jnp.dot(zrow.astype(jnp.bfloat16), wz_ref[...],
                preferred_element_type=jnp.float32) + bz_ref[...])
    s = jax.lax.dot_general(tb, z.astype(jnp.bfloat16),
                            (((1,), (1,)), ((), ())),
                            preferred_element_type=jnp.float32)
    e = jnp.exp(s)
    out_ref[0] = e / jnp.sum(e, axis=1, keepdims=True)


def kernel(txt_embeds, zone_embeds, W_txt, b_txt, W_zone, b_zone):
    bt = b_txt.reshape(1, _OUT)
    bz = b_zone.reshape(1, _OUT)
    return pl.pallas_call(
        _zoner_body,
        grid=(_B,),
        in_specs=[
            pl.BlockSpec((_B, _D), lambda b: (0, 0)),
            pl.BlockSpec(memory_space=pltpu.MemorySpace.HBM),
            pl.BlockSpec((_D, _OUT), lambda b: (0, 0)),
            pl.BlockSpec((1, _OUT), lambda b: (0, 0)),
            pl.BlockSpec((_D, _OUT), lambda b: (0, 0)),
            pl.BlockSpec((1, _OUT), lambda b: (0, 0)),
        ],
        out_specs=pl.BlockSpec((1, 1, _Z), lambda b: (b, 0, 0)),
        out_shape=jax.ShapeDtypeStruct((_B, 1, _Z), jnp.float32),
        scratch_shapes=[
            pltpu.VMEM((_B, _OUT), jnp.float32),
            pltpu.VMEM((_NBUF, _Z, _D), jnp.float32),
            pltpu.SemaphoreType.DMA((_NBUF,)),
        ],
    )(txt_embeds, zone_embeds,
      W_txt.astype(jnp.bfloat16), bt,
      W_zone.astype(jnp.bfloat16), bz).reshape(_B, _Z)


# speedup vs baseline: 1.5994x; 1.0026x over previous
import math

import jax
import jax.numpy as jnp
from jax.experimental import pallas as pl
from jax.experimental.pallas import tpu as pltpu

_B, _Z, _D, _OUT = 64, 1024, 1024, 256
_NBUF = 4


def _zoner_body(txt_ref, zone_hbm, wt_ref, bt_ref, wz_ref, bz_ref,
                out_ref, t_ref, zbuf, sems):
    b = pl.program_id(0)

    def _copy(row):
        slot = jax.lax.rem(row, _NBUF)
        return pltpu.make_async_copy(
            zone_hbm.at[pl.ds(row, 1)],
            zbuf.at[pl.ds(slot, 1)],
            sems.at[slot])

    @pl.when(b == 0)
    def _():
        for k in range(_NBUF - 1):
            _copy(k).start()
        t_ref[...] = jnp.tanh(
            jnp.dot(txt_ref[...].astype(jnp.bfloat16), wt_ref[...],
                    preferred_element_type=jnp.float32)
            + bt_ref[...]) * (1.0 / math.sqrt(_D))

    @pl.when(b + _NBUF - 1 < _B)
    def _():
        _copy(b + _NBUF - 1).start()

    _copy(b).wait()
    zrow = zbuf[pl.ds(jax.lax.rem(b, _NBUF), 1)][0]
    tb = t_ref[pl.ds(b, 1), :].astype(jnp.bfloat16)
    z = jnp.tanh(
        jnp.dot(zrow.astype(jnp.bfloat16), wz_ref[...],
                preferred_element_type=jnp.float32) + bz_ref[...])
    s = jax.lax.dot_general(tb, z.astype(jnp.bfloat16),
                            (((1,), (1,)), ((), ())),
                            preferred_element_type=jnp.float32)
    e = jnp.exp(s)
    out_ref[0] = e / jnp.sum(e, axis=1, keepdims=True)


def kernel(txt_embeds, zone_embeds, W_txt, b_txt, W_zone, b_zone):
    bt = b_txt.reshape(1, _OUT)
    bz = b_zone.reshape(1, _OUT)
    return pl.pallas_call(
        _zoner_body,
        grid=(_B,),
        in_specs=[
            pl.BlockSpec((_B, _D), lambda b: (0, 0)),
            pl.BlockSpec(memory_space=pltpu.MemorySpace.HBM),
            pl.BlockSpec((_D, _OUT), lambda b: (0, 0)),
            pl.BlockSpec((1, _OUT), lambda b: (0, 0)),
            pl.BlockSpec((_D, _OUT), lambda b: (0, 0)),
            pl.BlockSpec((1, _OUT), lambda b: (0, 0)),
        ],
        out_specs=pl.BlockSpec((1, 1, _Z), lambda b: (b, 0, 0)),
        out_shape=jax.ShapeDtypeStruct((_B, 1, _Z), jnp.float32),
        scratch_shapes=[
            pltpu.VMEM((_B, _OUT), jnp.float32),
            pltpu.VMEM((_NBUF, _Z, _D), jnp.float32),
            pltpu.SemaphoreType.DMA((_NBUF,)),
        ],
    )(txt_embeds, zone_embeds,
      W_txt.astype(jnp.bfloat16), bt,
      W_zone.astype(jnp.bfloat16), bz).reshape(_B, _Z)


# speedup vs baseline: 1.6368x; 1.0234x over previous
import math

import jax
import jax.numpy as jnp
from jax.experimental import pallas as pl
from jax.experimental.pallas import tpu as pltpu

_B, _Z, _D, _OUT = 64, 1024, 1024, 256
_NBUF = 4


def _zoner_body(txt_ref, zone_hbm, wt_ref, bt_ref, wz_ref, bz_ref,
                out_ref, t_ref, zbuf, sems):
    b = pl.program_id(0)

    def _copy(row):
        slot = jax.lax.rem(row, _NBUF)
        return pltpu.make_async_copy(
            zone_hbm.at[pl.ds(row, 1)],
            zbuf.at[pl.ds(slot, 1)],
            sems.at[slot])

    @pl.when(b == 0)
    def _():
        for k in range(_NBUF - 1):
            _copy(k).start()
        t_ref[...] = jnp.tanh(
            jnp.dot(txt_ref[...].astype(jnp.bfloat16), wt_ref[...],
                    preferred_element_type=jnp.float32)
            + bt_ref[...]) * (1.0 / math.sqrt(_D))

    @pl.when(b + _NBUF - 1 < _B)
    def _():
        _copy(b + _NBUF - 1).start()

    _copy(b).wait()
    zrow = zbuf[pl.ds(jax.lax.rem(b, _NBUF), 1)][0]
    tb = t_ref[pl.ds(b, 1), :].astype(jnp.bfloat16)
    z = jnp.tanh(
        jnp.dot(zrow, wz_ref[...],
                preferred_element_type=jnp.float32) + bz_ref[...])
    s = jax.lax.dot_general(tb, z.astype(jnp.bfloat16),
                            (((1,), (1,)), ((), ())),
                            preferred_element_type=jnp.float32)
    e = jnp.exp(s)
    out_ref[0] = e / jnp.sum(e, axis=1, keepdims=True)


def kernel(txt_embeds, zone_embeds, W_txt, b_txt, W_zone, b_zone):
    bt = b_txt.reshape(1, _OUT)
    bz = b_zone.reshape(1, _OUT)
    return pl.pallas_call(
        _zoner_body,
        grid=(_B,),
        in_specs=[
            pl.BlockSpec((_B, _D), lambda b: (0, 0)),
            pl.BlockSpec(memory_space=pltpu.MemorySpace.HBM),
            pl.BlockSpec((_D, _OUT), lambda b: (0, 0)),
            pl.BlockSpec((1, _OUT), lambda b: (0, 0)),
            pl.BlockSpec((_D, _OUT), lambda b: (0, 0)),
            pl.BlockSpec((1, _OUT), lambda b: (0, 0)),
        ],
        out_specs=pl.BlockSpec((1, 1, _Z), lambda b: (b, 0, 0)),
        out_shape=jax.ShapeDtypeStruct((_B, 1, _Z), jnp.float32),
        scratch_shapes=[
            pltpu.VMEM((_B, _OUT), jnp.float32),
            pltpu.VMEM((_NBUF, _Z, _D), jnp.float32),
            pltpu.SemaphoreType.DMA((_NBUF,)),
        ],
    )(txt_embeds, zone_embeds,
      W_txt.astype(jnp.bfloat16), bt,
      W_zone, bz).reshape(_B, _Z)
